# baseline (speedup 1.0000x reference)
import jax
import jax.numpy as jnp
from jax import lax
from jax.experimental import pallas as pl
from jax.experimental.pallas import tpu as pltpu
from jax.experimental.pallas import tpu_sc as plsc

_N_ROWS = 50000
_D = 256
_NL = _D // 16
_SEGS = 1024
_T = 80
_NC = 2
_NS = 16
_NW = _NC * _NS
_HI = 17
_OUTROWS = _SEGS + 24
_CW = 128


def _sc_body(h_hbm, ids_hbm, outab_hbm, cnts_hbm, psum_hbm, pseg_hbm,
             pcnt_hbm, rows_v, ids_v, fbuf_v, cbuf_v, psum_v, pseg_v, pcnt_v,
             rsem, isem):
    c = lax.axis_index("c")
    s = lax.axis_index("s")
    w = s * _NC + c
    lanes = lax.iota(jnp.int32, 16)
    zeros16 = jnp.zeros((16,), jnp.float32)
    ones16 = jnp.ones((16,), jnp.float32)

    def _zero_bufs(i, _):
        for j in range(_NL):
            fbuf_v[i, pl.ds(j * 16, 16)] = zeros16
        for j in range(_CW // 16):
            cbuf_v[i, pl.ds(j * 16, 16)] = zeros16
        return 0

    lax.fori_loop(0, 32, _zero_bufs, 0)
    zbase = c * _OUTROWS + s * 64
    for k in range(4):
        pltpu.sync_copy(fbuf_v.at[pl.ds(0, 16)], outab_hbm.at[zbase + k * 16 + lanes])
        pltpu.sync_copy(cbuf_v.at[pl.ds(0, 16)], cnts_hbm.at[zbase + k * 16 + lanes])

    @pl.when(s == 0)
    def _():
        sbase = c * _OUTROWS + _SEGS + 1
        pltpu.sync_copy(fbuf_v.at[pl.ds(0, 16)], outab_hbm.at[sbase + lanes])
        pltpu.sync_copy(cbuf_v.at[pl.ds(0, 16)], cnts_hbm.at[sbase + lanes])

    plsc.subcore_barrier()

    n_tiles = jnp.where(w < _HI, 20, 19)
    n_rows = n_tiles * _T
    row0 = _T * (19 * w + jnp.minimum(w, _HI))
    dump = c * _OUTROWS + _SEGS

    pltpu.sync_copy(ids_hbm.at[pl.ds(row0, 16)], ids_v.at[pl.ds(2 * _T, 16)])
    seg0 = ids_v[pl.ds(2 * _T, 16)][0]

    pltpu.async_copy(ids_hbm.at[pl.ds(row0, _T)], ids_v.at[pl.ds(0, _T)],
                     isem)
    pltpu.async_copy(h_hbm.at[pl.ds(row0, _T)], rows_v.at[pl.ds(0, _T)],
                     rsem)

    def _rowgroup(rg, carry):
        g8 = rg * 8
        rloc8 = lax.rem(g8, jnp.int32(2 * _T))
        boundary = lax.rem(g8, jnp.int32(_T)) == 0

        @pl.when(boundary)
        def _():
            pltpu.make_async_copy(ids_hbm.at[pl.ds(0, _T)],
                                  ids_v.at[pl.ds(0, _T)], isem).wait()
            pltpu.make_async_copy(h_hbm.at[pl.ds(0, _T)],
                                  rows_v.at[pl.ds(0, _T)], rsem).wait()

        ti = lax.div(g8, jnp.int32(_T))

        @pl.when(jnp.logical_and(boundary, ti + 1 < n_tiles))
        def _():
            nbase = row0 + (ti + 1) * _T
            nhalf = lax.rem(ti + 1, jnp.int32(2)) * _T
            pltpu.async_copy(ids_hbm.at[pl.ds(nbase, _T)],
                             ids_v.at[pl.ds(nhalf, _T)], isem)
            pltpu.async_copy(h_hbm.at[pl.ds(nbase, _T)],
                             rows_v.at[pl.ds(nhalf, _T)], rsem)

        slot0, idx_vec0, idx_vec20, _cf, _sn, _pv, _tt = carry

        @pl.when(slot0 >= 16)
        def _():
            pltpu.sync_copy(fbuf_v.at[pl.ds(0, 16)], outab_hbm.at[idx_vec0])
            pltpu.sync_copy(cbuf_v.at[pl.ds(0, 16)], cnts_hbm.at[idx_vec0])
            pltpu.sync_copy(fbuf_v.at[pl.ds(16, 16)], outab_hbm.at[idx_vec20])
            pltpu.sync_copy(cbuf_v.at[pl.ds(16, 16)], cnts_hbm.at[idx_vec20])

        gdrained = slot0 >= 16
        dumpv = jnp.full((16,), dump, jnp.int32)
        carry = (jnp.where(gdrained, 0, slot0),
                 jnp.where(gdrained, dumpv, idx_vec0),
                 jnp.where(gdrained, dumpv, idx_vec20),
                 _cf, _sn, _pv, _tt)

        idv = ids_v[pl.ds(rloc8, 16)]
        for j in range(8):
            slot, idx_vec, idx_vec2, cnt_f, seen, prev, total = carry
            rloc = rloc8 + j
            seg = idv[j]
            changed_now = seg != prev

            target = jnp.where(seen, c * _OUTROWS + prev,
                               c * _OUTROWS + _SEGS + 1 + s)
            sel_slot = jnp.where(changed_now, slot, jnp.int32(-1))
            idx_vec = jnp.where(lanes == sel_slot, target, idx_vec)
            slot = slot + changed_now.astype(jnp.int32)

            @pl.when(slot >= 16)
            def _():
                pltpu.sync_copy(fbuf_v.at[pl.ds(0, 16)], outab_hbm.at[idx_vec])
                pltpu.sync_copy(cbuf_v.at[pl.ds(0, 16)], cnts_hbm.at[idx_vec])

            drained = slot >= 16
            slot = jnp.where(drained, 0, slot)
            idx_vec = jnp.where(drained,
                                jnp.full((16,), dump, jnp.int32), idx_vec)
            seen = jnp.logical_or(seen, changed_now)
            cnt_f = jnp.where(changed_now, 0.0, cnt_f) + 1.0
            total = tuple(
                jnp.where(changed_now, zeros16, total[q])
                + rows_v[rloc, pl.ds(q * 16, 16)]
                for q in range(_NL))

            cbuf_v[slot, pl.ds(0, 16)] = jnp.full((16,), cnt_f, jnp.float32)
            for q in range(_NL):
                fbuf_v[slot, pl.ds(q * 16, 16)] = total[q]

            carry = (slot, idx_vec, idx_vec2, cnt_f, seen, seg, total)
        return carry

    init = (jnp.int32(0),
            jnp.full((16,), dump, jnp.int32),
            jnp.full((16,), dump, jnp.int32),
            jnp.float32(0.0),
            jnp.bool_(False),
            seg0,
            tuple(zeros16 for _ in range(_NL)))

    slot, idx_vec, idx_vec2, cnt_f, seen, prev, total = lax.fori_loop(
        0, n_rows // 8, _rowgroup, init)

    pltpu.sync_copy(fbuf_v.at[pl.ds(0, 16)], outab_hbm.at[idx_vec])
    pltpu.sync_copy(cbuf_v.at[pl.ds(0, 16)], cnts_hbm.at[idx_vec])
    pltpu.sync_copy(fbuf_v.at[pl.ds(16, 16)], outab_hbm.at[idx_vec2])
    pltpu.sync_copy(cbuf_v.at[pl.ds(16, 16)], cnts_hbm.at[idx_vec2])

    pseg_v[pl.ds(0, 16)] = jnp.full((16,), seg0, jnp.int32)
    pcnt_v[pl.ds(0, 16)] = jnp.full((16,), 0.0, jnp.float32)
    pseg_v[pl.ds(16, 16)] = jnp.full((16,), prev, jnp.int32)
    pcnt_v[pl.ds(16, 16)] = jnp.full((16,), cnt_f, jnp.float32)
    for j in range(_NL):
        psum_v[0, pl.ds(j * 16, 16)] = zeros16 - zeros16
        psum_v[1, pl.ds(j * 16, 16)] = total[j] - zeros16

    pltpu.sync_copy(psum_v, psum_hbm.at[pl.ds(w * 8, 8)])
    pltpu.sync_copy(pseg_v, pseg_hbm.at[pl.ds(w * 32, 32)])
    pltpu.sync_copy(pcnt_v, pcnt_hbm.at[pl.ds(w * 32, 32)])


_sc_pool = pl.kernel(
    _sc_body,
    out_type=(
        jax.ShapeDtypeStruct((_NC * _OUTROWS, _D), jnp.float32),
        jax.ShapeDtypeStruct((_NC * _OUTROWS, _CW), jnp.float32),
        jax.ShapeDtypeStruct((_NW * 8, _D), jnp.float32),
        jax.ShapeDtypeStruct((_NW * 32,), jnp.int32),
        jax.ShapeDtypeStruct((_NW * 32,), jnp.float32),
    ),
    mesh=plsc.VectorSubcoreMesh(core_axis_name="c", subcore_axis_name="s",
                                num_cores=_NC, num_subcores=_NS),
    scratch_types=[
        pltpu.VMEM((2 * _T, _D), jnp.float32),
        pltpu.VMEM((2 * _T + 16,), jnp.int32),
        pltpu.VMEM((32, _D), jnp.float32),
        pltpu.VMEM((32, _CW), jnp.float32),
        pltpu.VMEM((8, _D), jnp.float32),
        pltpu.VMEM((32,), jnp.int32),
        pltpu.VMEM((32,), jnp.float32),
        pltpu.SemaphoreType.DMA,
        pltpu.SemaphoreType.DMA,
    ],
)


def _fix_body(outab_ref, cnts_ref, psum_ref, pseg_ref, pcnt_ref, o_ref,
              cnt_sc):
    o_ref[...] = (outab_ref[0:_SEGS, :]
                  + outab_ref[_OUTROWS:_OUTROWS + _SEGS, :])
    cnt_sc[...] = (cnts_ref[0:_SEGS, 0:1]
                   + cnts_ref[_OUTROWS:_OUTROWS + _SEGS, 0:1])

    def _entry(k, _):
        srow = (k % 2) * _OUTROWS + _SEGS + 1 + k // 2
        sl = pseg_ref[k * 32]
        o_ref[pl.ds(sl, 1), :] = (o_ref[pl.ds(sl, 1), :]
                                  + outab_ref[pl.ds(srow, 1), :])
        cnt_sc[pl.ds(sl, 1), :] = (cnt_sc[pl.ds(sl, 1), :]
                                   + cnts_ref[pl.ds(srow, 1), 0:1])
        st = pseg_ref[k * 32 + 16]
        o_ref[pl.ds(st, 1), :] = (o_ref[pl.ds(st, 1), :]
                                  + psum_ref[pl.ds(k * 8 + 1, 1), :])
        cnt_sc[pl.ds(st, 1), :] = (cnt_sc[pl.ds(st, 1), :]
                                   + pcnt_ref[k * 32 + 16])
        return 0

    lax.fori_loop(0, _NW, _entry, 0)
    o_ref[...] = o_ref[...] / jnp.maximum(cnt_sc[...], 1.0)


_fixup = pl.pallas_call(
    _fix_body,
    in_specs=[
        pl.BlockSpec(memory_space=pltpu.VMEM),
        pl.BlockSpec(memory_space=pltpu.VMEM),
        pl.BlockSpec(memory_space=pltpu.VMEM),
        pl.BlockSpec(memory_space=pltpu.SMEM),
        pl.BlockSpec(memory_space=pltpu.SMEM),
    ],
    out_shape=jax.ShapeDtypeStruct((_SEGS, _D), jnp.float32),
    scratch_shapes=[pltpu.VMEM((_SEGS, 1), jnp.float32)],
)


def kernel(h, segment_ids, num_segments):
    del num_segments
    ids = segment_ids.astype(jnp.int32)
    outab, cnts, psum, pseg, pcnt = _sc_pool(h, ids)
    return _fixup(outab, cnts, psum, pseg, pcnt)


# speedup vs baseline: 1.1900x; 1.1900x over previous
import jax
import jax.numpy as jnp
from jax import lax
from jax.experimental import pallas as pl
from jax.experimental.pallas import tpu as pltpu
from jax.experimental.pallas import tpu_sc as plsc

_N_ROWS = 50000
_D = 256
_NL = _D // 16
_SEGS = 1024
_T = 80
_NC = 2
_NS = 16
_NW = _NC * _NS
_HI = 17
_OUTROWS = _SEGS + 24
_CW = 128


def _sc_body(h_hbm, ids_hbm, outab_hbm, cnts_hbm, psum_hbm, pseg_hbm,
             pcnt_hbm, rows_v, ids_v, fbuf_v, cbuf_v, psum_v, pseg_v, pcnt_v,
             rsem, isem):
    c = lax.axis_index("c")
    s = lax.axis_index("s")
    w = s * _NC + c
    lanes = lax.iota(jnp.int32, 16)
    zeros16 = jnp.zeros((16,), jnp.float32)
    ones16 = jnp.ones((16,), jnp.float32)

    def _zero_bufs(i, _):
        for j in range(_NL):
            fbuf_v[i, pl.ds(j * 16, 16)] = zeros16
        for j in range(_CW // 16):
            cbuf_v[i, pl.ds(j * 16, 16)] = zeros16
        return 0

    lax.fori_loop(0, 16, _zero_bufs, 0)
    zbase = c * _OUTROWS + s * 64
    for k in range(4):
        pltpu.sync_copy(fbuf_v, outab_hbm.at[zbase + k * 16 + lanes])
        pltpu.sync_copy(cbuf_v, cnts_hbm.at[zbase + k * 16 + lanes])

    @pl.when(s == 0)
    def _():
        sbase = c * _OUTROWS + _SEGS + 1
        pltpu.sync_copy(fbuf_v, outab_hbm.at[sbase + lanes])
        pltpu.sync_copy(cbuf_v, cnts_hbm.at[sbase + lanes])

    plsc.subcore_barrier()

    n_tiles = jnp.where(w < _HI, 20, 19)
    n_rows = n_tiles * _T
    row0 = _T * (19 * w + jnp.minimum(w, _HI))
    dump = c * _OUTROWS + _SEGS

    pltpu.sync_copy(ids_hbm.at[pl.ds(row0, 16)], ids_v.at[pl.ds(2 * _T, 16)])
    seg0 = ids_v[pl.ds(2 * _T, 16)][0]

    pltpu.async_copy(ids_hbm.at[pl.ds(row0, _T)], ids_v.at[pl.ds(0, _T)],
                     isem)
    pltpu.async_copy(h_hbm.at[pl.ds(row0, _T)], rows_v.at[pl.ds(0, _T)],
                     rsem)

    def _rowgroup(rg, carry):
        g8 = rg * 16
        rloc8 = lax.rem(g8, jnp.int32(2 * _T))
        boundary = lax.rem(g8, jnp.int32(_T)) == 0

        @pl.when(boundary)
        def _():
            pltpu.make_async_copy(ids_hbm.at[pl.ds(0, _T)],
                                  ids_v.at[pl.ds(0, _T)], isem).wait()
            pltpu.make_async_copy(h_hbm.at[pl.ds(0, _T)],
                                  rows_v.at[pl.ds(0, _T)], rsem).wait()

        ti = lax.div(g8, jnp.int32(_T))

        @pl.when(jnp.logical_and(boundary, ti + 1 < n_tiles))
        def _():
            nbase = row0 + (ti + 1) * _T
            nhalf = lax.rem(ti + 1, jnp.int32(2)) * _T
            pltpu.async_copy(ids_hbm.at[pl.ds(nbase, _T)],
                             ids_v.at[pl.ds(nhalf, _T)], isem)
            pltpu.async_copy(h_hbm.at[pl.ds(nbase, _T)],
                             rows_v.at[pl.ds(nhalf, _T)], rsem)

        idv = ids_v[pl.ds(rloc8, 16)]
        for j in range(16):
            slot, idx_vec, cnt_f, seen, prev, total = carry
            rloc = rloc8 + j
            seg = idv[j]
            changed_now = seg != prev

            target = jnp.where(seen, c * _OUTROWS + prev,
                               c * _OUTROWS + _SEGS + 1 + s)
            sel_slot = jnp.where(changed_now, slot, jnp.int32(-1))
            idx_vec = jnp.where(lanes == sel_slot, target, idx_vec)
            slot = slot + changed_now.astype(jnp.int32)

            @pl.when(slot >= 16)
            def _():
                pltpu.sync_copy(fbuf_v, outab_hbm.at[idx_vec])
                pltpu.sync_copy(cbuf_v, cnts_hbm.at[idx_vec])

            drained = slot >= 16
            slot = jnp.where(drained, 0, slot)
            idx_vec = jnp.where(drained,
                                jnp.full((16,), dump, jnp.int32), idx_vec)
            seen = jnp.logical_or(seen, changed_now)
            cnt_f = jnp.where(changed_now, 0.0, cnt_f) + 1.0
            total = tuple(
                jnp.where(changed_now, zeros16, total[q])
                + rows_v[rloc, pl.ds(q * 16, 16)]
                for q in range(_NL))

            cbuf_v[slot, pl.ds(0, 16)] = jnp.full((16,), cnt_f, jnp.float32)
            for q in range(_NL):
                fbuf_v[slot, pl.ds(q * 16, 16)] = total[q]

            carry = (slot, idx_vec, cnt_f, seen, seg, total)
        return carry

    init = (jnp.int32(0),
            jnp.full((16,), dump, jnp.int32),
            jnp.float32(0.0),
            jnp.bool_(False),
            seg0,
            tuple(zeros16 for _ in range(_NL)))

    slot, idx_vec, cnt_f, seen, prev, total = lax.fori_loop(
        0, n_rows // 16, _rowgroup, init)

    pltpu.sync_copy(fbuf_v, outab_hbm.at[idx_vec])
    pltpu.sync_copy(cbuf_v, cnts_hbm.at[idx_vec])

    pseg_v[pl.ds(0, 16)] = jnp.full((16,), seg0, jnp.int32)
    pcnt_v[pl.ds(0, 16)] = jnp.full((16,), 0.0, jnp.float32)
    pseg_v[pl.ds(16, 16)] = jnp.full((16,), prev, jnp.int32)
    pcnt_v[pl.ds(16, 16)] = jnp.full((16,), cnt_f, jnp.float32)
    for j in range(_NL):
        psum_v[0, pl.ds(j * 16, 16)] = zeros16 - zeros16
        psum_v[1, pl.ds(j * 16, 16)] = total[j] - zeros16

    pltpu.sync_copy(psum_v, psum_hbm.at[pl.ds(w * 8, 8)])
    pltpu.sync_copy(pseg_v, pseg_hbm.at[pl.ds(w * 32, 32)])
    pltpu.sync_copy(pcnt_v, pcnt_hbm.at[pl.ds(w * 32, 32)])


_sc_pool = pl.kernel(
    _sc_body,
    out_type=(
        jax.ShapeDtypeStruct((_NC * _OUTROWS, _D), jnp.float32),
        jax.ShapeDtypeStruct((_NC * _OUTROWS, _CW), jnp.float32),
        jax.ShapeDtypeStruct((_NW * 8, _D), jnp.float32),
        jax.ShapeDtypeStruct((_NW * 32,), jnp.int32),
        jax.ShapeDtypeStruct((_NW * 32,), jnp.float32),
    ),
    mesh=plsc.VectorSubcoreMesh(core_axis_name="c", subcore_axis_name="s",
                                num_cores=_NC, num_subcores=_NS),
    scratch_types=[
        pltpu.VMEM((2 * _T, _D), jnp.float32),
        pltpu.VMEM((2 * _T + 16,), jnp.int32),
        pltpu.VMEM((16, _D), jnp.float32),
        pltpu.VMEM((16, _CW), jnp.float32),
        pltpu.VMEM((8, _D), jnp.float32),
        pltpu.VMEM((32,), jnp.int32),
        pltpu.VMEM((32,), jnp.float32),
        pltpu.SemaphoreType.DMA,
        pltpu.SemaphoreType.DMA,
    ],
)


def _fix_body(outab_ref, cnts_ref, psum_ref, pseg_ref, pcnt_ref, o_ref,
              cnt_sc):
    o_ref[...] = (outab_ref[0:_SEGS, :]
                  + outab_ref[_OUTROWS:_OUTROWS + _SEGS, :])
    cnt_sc[...] = (cnts_ref[0:_SEGS, 0:1]
                   + cnts_ref[_OUTROWS:_OUTROWS + _SEGS, 0:1])

    def _entry(k, _):
        srow = (k % 2) * _OUTROWS + _SEGS + 1 + k // 2
        sl = pseg_ref[k * 32]
        o_ref[pl.ds(sl, 1), :] = (o_ref[pl.ds(sl, 1), :]
                                  + outab_ref[pl.ds(srow, 1), :])
        cnt_sc[pl.ds(sl, 1), :] = (cnt_sc[pl.ds(sl, 1), :]
                                   + cnts_ref[pl.ds(srow, 1), 0:1])
        st = pseg_ref[k * 32 + 16]
        o_ref[pl.ds(st, 1), :] = (o_ref[pl.ds(st, 1), :]
                                  + psum_ref[pl.ds(k * 8 + 1, 1), :])
        cnt_sc[pl.ds(st, 1), :] = (cnt_sc[pl.ds(st, 1), :]
                                   + pcnt_ref[k * 32 + 16])
        return 0

    lax.fori_loop(0, _NW, _entry, 0)
    o_ref[...] = o_ref[...] / jnp.maximum(cnt_sc[...], 1.0)


_fixup = pl.pallas_call(
    _fix_body,
    in_specs=[
        pl.BlockSpec(memory_space=pltpu.VMEM),
        pl.BlockSpec(memory_space=pltpu.VMEM),
        pl.BlockSpec(memory_space=pltpu.VMEM),
        pl.BlockSpec(memory_space=pltpu.SMEM),
        pl.BlockSpec(memory_space=pltpu.SMEM),
    ],
    out_shape=jax.ShapeDtypeStruct((_SEGS, _D), jnp.float32),
    scratch_shapes=[pltpu.VMEM((_SEGS, 1), jnp.float32)],
)


def kernel(h, segment_ids, num_segments):
    del num_segments
    ids = segment_ids.astype(jnp.int32)
    outab, cnts, psum, pseg, pcnt = _sc_pool(h, ids)
    return _fixup(outab, cnts, psum, pseg, pcnt)
